# baseline (speedup 1.0000x reference)
import functools

import jax
import jax.numpy as jnp
from jax import lax
from jax.experimental import pallas as pl
from jax.experimental.pallas import tpu as pltpu
from jax.experimental.pallas import tpu_sc as plsc

L = 64
N = 16384
D = 16
B = 8

NC = 2
NS = 16
NW = NC * NS
CHUNK = N // NW
ROW = CHUNK * D
LANES = 16
GRP = ROW // LANES


def _body(hist_hbm, value_hbm, idxtab_hbm, coef_hbm, out_hbm,
          idx_v, coef_v, val_v, d0b, d1b, ob, sem):
    wid = lax.axis_index("s") * NC + lax.axis_index("c")

    pltpu.sync_copy(idxtab_hbm.at[wid], idx_v)
    pltpu.sync_copy(coef_hbm, coef_v)
    pltpu.sync_copy(value_hbm.at[wid], val_v)

    for b in range(B):
        pltpu.async_copy(hist_hbm.at[idx_v.at[b]], d0b, sem).wait()
        pltpu.async_copy(hist_hbm.at[idx_v.at[B + b]], d1b, sem).wait()

        ca = coef_v[b, :]
        cc = coef_v[B + b, :]
        cv = coef_v[2 * B + b, :]

        def compute(i, carry):
            a = d0b[0, pl.ds(i * LANES, LANES)]
            c = d1b[0, pl.ds(i * LANES, LANES)]
            v = val_v[pl.ds(i * LANES, LANES)]
            ob[pl.ds(i * LANES, LANES)] = ca * a + cc * c + cv * v
            return carry

        lax.fori_loop(0, GRP, compute, 0, unroll=8)

        pltpu.sync_copy(ob, out_hbm.at[b, wid])


@jax.jit
def _sc_delay(hist2, value2, idxtab, coef):
    call = functools.partial(
        pl.kernel,
        mesh=plsc.VectorSubcoreMesh(core_axis_name="c", subcore_axis_name="s"),
        out_type=jax.ShapeDtypeStruct((B, NW, ROW), jnp.float32),
        scratch_types=[
            pltpu.VMEM((2 * B, 1), jnp.int32),
            pltpu.VMEM((3 * B, LANES), jnp.float32),
            pltpu.VMEM((ROW,), jnp.float32),
            pltpu.VMEM((1, ROW), jnp.float32),
            pltpu.VMEM((1, ROW), jnp.float32),
            pltpu.VMEM((ROW,), jnp.float32),
            pltpu.SemaphoreType.DMA,
        ],
    )(_body)
    return call(hist2, value2, idxtab, coef)


def kernel(history, value, delay_frac, write_idx, delay_int):
    hist2 = history.reshape(L * NW, ROW)
    value2 = value.reshape(NW, ROW)

    wi = jnp.asarray(write_idx, jnp.int32)
    i0 = jnp.mod(wi - delay_int, L)
    i1 = jnp.mod(wi - delay_int - 1, L)
    wrow = jnp.mod(wi, L)
    w = delay_frac.astype(jnp.float32)
    m0 = (i0 == wrow).astype(jnp.float32)
    m1 = (i1 == wrow).astype(jnp.float32)
    ca = (1.0 - w) * (1.0 - m0)
    cc = w * (1.0 - m1)
    cv = (1.0 - w) * m0 + w * m1
    coef = jnp.broadcast_to(
        jnp.concatenate([ca, cc, cv])[:, None], (3 * B, LANES))

    wids = jnp.arange(NW, dtype=jnp.int32)[:, None]
    rows = jnp.concatenate([i0, i1]).astype(jnp.int32)[None, :]
    idxtab = (rows * NW + wids)[..., None]

    out3 = _sc_delay(hist2, value2, idxtab, coef)
    return out3.reshape(B, N, D)


# speedup vs baseline: 1.2589x; 1.2589x over previous
import functools

import jax
import jax.numpy as jnp
from jax import lax
from jax.experimental import pallas as pl
from jax.experimental.pallas import tpu as pltpu
from jax.experimental.pallas import tpu_sc as plsc

L = 64
N = 16384
D = 16
B = 8

NC = 2
NS = 16
NW = NC * NS
CHUNK = N // NW
ROW = CHUNK * D
LANES = 16
SLR = ROW // 128
SUBG = 128 // LANES


def _body(hist_hbm, value_hbm, idxtab_hbm, coef_hbm, out_hbm,
          idx_v, coef_v, val_v, d0b, d1b, ob, sem):
    wid = lax.axis_index("s") * NC + lax.axis_index("c")

    pltpu.sync_copy(idxtab_hbm.at[wid], idx_v)
    pltpu.sync_copy(coef_hbm, coef_v)
    pltpu.sync_copy(value_hbm.at[wid], val_v)

    for b in range(B):
        pltpu.async_copy(hist_hbm.at[idx_v.at[b]], d0b, sem).wait()
        pltpu.async_copy(hist_hbm.at[idx_v.at[B + b]], d1b, sem).wait()

        ca = coef_v[b, :]
        cc = coef_v[B + b, :]
        cv = coef_v[2 * B + b, :]

        def compute(i, carry):
            for j in range(SUBG):
                sl = pl.ds(j * LANES, LANES)
                a = d0b[0, i, sl]
                c = d1b[0, i, sl]
                v = val_v[i, sl]
                ob[i, sl] = ca * a + cc * c + cv * v
            return carry

        lax.fori_loop(0, SLR, compute, 0)

        pltpu.sync_copy(ob, out_hbm.at[b, wid])


@jax.jit
def _sc_delay(hist3, value3, idxtab, coef):
    call = functools.partial(
        pl.kernel,
        mesh=plsc.VectorSubcoreMesh(core_axis_name="c", subcore_axis_name="s"),
        out_type=jax.ShapeDtypeStruct((B, NW, SLR, 128), jnp.float32),
        scratch_types=[
            pltpu.VMEM((2 * B, 1), jnp.int32),
            pltpu.VMEM((3 * B, LANES), jnp.float32),
            pltpu.VMEM((SLR, 128), jnp.float32),
            pltpu.VMEM((1, SLR, 128), jnp.float32),
            pltpu.VMEM((1, SLR, 128), jnp.float32),
            pltpu.VMEM((SLR, 128), jnp.float32),
            pltpu.SemaphoreType.DMA,
        ],
    )(_body)
    return call(hist3, value3, idxtab, coef)


def kernel(history, value, delay_frac, write_idx, delay_int):
    hist3 = history.reshape(L * NW, SLR, 128)
    value3 = value.reshape(NW, SLR, 128)

    wi = jnp.asarray(write_idx, jnp.int32)
    i0 = jnp.mod(wi - delay_int, L)
    i1 = jnp.mod(wi - delay_int - 1, L)
    wrow = jnp.mod(wi, L)
    w = delay_frac.astype(jnp.float32)
    m0 = (i0 == wrow).astype(jnp.float32)
    m1 = (i1 == wrow).astype(jnp.float32)
    ca = (1.0 - w) * (1.0 - m0)
    cc = w * (1.0 - m1)
    cv = (1.0 - w) * m0 + w * m1
    coef = jnp.broadcast_to(
        jnp.concatenate([ca, cc, cv])[:, None], (3 * B, LANES))

    wids = jnp.arange(NW, dtype=jnp.int32)[:, None]
    rows = jnp.concatenate([i0, i1]).astype(jnp.int32)[None, :]
    idxtab = (rows * NW + wids)[..., None]

    out4 = _sc_delay(hist3, value3, idxtab, coef)
    return out4.reshape(B, N, D)


# speedup vs baseline: 1.2870x; 1.0223x over previous
import functools

import jax
import jax.numpy as jnp
from jax import lax
from jax.experimental import pallas as pl
from jax.experimental.pallas import tpu as pltpu
from jax.experimental.pallas import tpu_sc as plsc

L = 64
N = 16384
D = 16
B = 8

NC = 2
NS = 16
NW = NC * NS
CHUNK = N // NW
LANES = 16
HALVES = 2
HR = (CHUNK * D) // (HALVES * 128)
SUBG = 128 // LANES


def _body(hist_hbm, value_hbm, dint_hbm, widx_hbm, coef_hbm, out_hbm,
          dint_v, widx_v, coef_v, idxr, val_v, d0h, d1h, ob0, ob1,
          sg0, sg1, so0, so1):
    wid = lax.axis_index("s") * NC + lax.axis_index("c")

    pltpu.sync_copy(dint_hbm, dint_v)
    pltpu.sync_copy(widx_hbm, widx_v)
    pltpu.sync_copy(coef_hbm, coef_v)
    pltpu.sync_copy(value_hbm.at[wid], val_v)

    dv = dint_v[...]
    wv = widx_v[...]
    lane = lax.broadcasted_iota(jnp.int32, (LANES,), 0)
    off = jnp.where(lane < B, wv - dv, wv - dv - 1)
    rowv = lax.rem(lax.rem(off, L) + L, L)
    base = (rowv * NW + wid) * HALVES
    idxr[pl.ds(0, LANES)] = base
    idxr[pl.ds(LANES, LANES)] = base + 1

    obufs = (ob0, ob1)
    osems = (so0, so1)
    out_pending = [None, None]

    for h in range(HALVES):
        g0 = pltpu.async_copy(hist_hbm.at[idxr.at[pl.ds(h * LANES, B)]], d0h, sg0)
        g1 = pltpu.async_copy(hist_hbm.at[idxr.at[pl.ds(h * LANES + B, B)]], d1h, sg1)
        g0.wait()
        g1.wait()

        for b in range(B):
            bs = pl.ds(b * LANES, LANES)
            ca = coef_v[0, bs]
            cc = coef_v[1, bs]
            cv = coef_v[2, bs]
            ob = obufs[b % 2]

            if out_pending[b % 2] is not None:
                out_pending[b % 2].wait()

            def compute(i, carry):
                for j in range(SUBG):
                    sl = pl.ds(j * LANES, LANES)
                    a = d0h[b, i, sl]
                    c = d1h[b, i, sl]
                    v = val_v[h * HR + i, sl]
                    ob[i, sl] = ca * a + cc * c + cv * v
                return carry

            lax.fori_loop(0, HR, compute, 0)

            out_pending[b % 2] = pltpu.async_copy(
                ob, out_hbm.at[b, wid, h], osems[b % 2])

        if h == 0:
            for p in range(2):
                if out_pending[p] is not None:
                    out_pending[p].wait()
                    out_pending[p] = None

    for p in range(2):
        if out_pending[p] is not None:
            out_pending[p].wait()


@jax.jit
def _sc_delay(hist4, value4, dint, widx, coef):
    call = functools.partial(
        pl.kernel,
        mesh=plsc.VectorSubcoreMesh(core_axis_name="c", subcore_axis_name="s"),
        compiler_params=pltpu.CompilerParams(use_tc_tiling_on_sc=True),
        out_type=jax.ShapeDtypeStruct((B, NW, HALVES, HR, 128), jnp.float32),
        scratch_types=[
            pltpu.VMEM((LANES,), jnp.int32),
            pltpu.VMEM((LANES,), jnp.int32),
            pltpu.VMEM((8, 128), jnp.float32),
            pltpu.VMEM((2 * LANES,), jnp.int32),
            pltpu.VMEM((HALVES * HR, 128), jnp.float32),
            pltpu.VMEM((B, HR, 128), jnp.float32),
            pltpu.VMEM((B, HR, 128), jnp.float32),
            pltpu.VMEM((HR, 128), jnp.float32),
            pltpu.VMEM((HR, 128), jnp.float32),
            pltpu.SemaphoreType.DMA,
            pltpu.SemaphoreType.DMA,
            pltpu.SemaphoreType.DMA,
            pltpu.SemaphoreType.DMA,
        ],
    )(_body)
    return call(hist4, value4, dint, widx, coef)


def kernel(history, value, delay_frac, write_idx, delay_int):
    hist4 = history.reshape(L * NW * HALVES, HR, 128)
    value4 = value.reshape(NW, HALVES * HR, 128)

    wi = jnp.asarray(write_idx, jnp.int32)
    i0 = jnp.mod(wi - delay_int, L)
    i1 = jnp.mod(wi - delay_int - 1, L)
    wrow = jnp.mod(wi, L)
    w = delay_frac.astype(jnp.float32)
    m0 = (i0 == wrow).astype(jnp.float32)
    m1 = (i1 == wrow).astype(jnp.float32)
    ca = (1.0 - w) * (1.0 - m0)
    cc = w * (1.0 - m1)
    cv = (1.0 - w) * m0 + w * m1
    coef = jnp.zeros((8, 128), jnp.float32)
    coef = coef.at[0].set(jnp.repeat(ca, LANES))
    coef = coef.at[1].set(jnp.repeat(cc, LANES))
    coef = coef.at[2].set(jnp.repeat(cv, LANES))

    dint = jnp.concatenate([delay_int, delay_int]).astype(jnp.int32)
    widx = jnp.full((LANES,), wi, dtype=jnp.int32)

    out5 = _sc_delay(hist4, value4, dint, widx, coef)
    return out5.reshape(B, N, D)


# speedup vs baseline: 17.0286x; 13.2310x over previous
import functools

import jax
import jax.numpy as jnp
from jax import lax
from jax.experimental import pallas as pl
from jax.experimental.pallas import tpu as pltpu
from jax.experimental.pallas import tpu_sc as plsc

L = 64
N = 16384
D = 16
B = 8

NC = 2
NS = 16
NW = NC * NS
CHUNK = N // NW
LANES = 16
GPR = CHUNK // LANES
LBITS = 6


def _body(histT_hbm, valueT_hbm, pack_hbm, outT_hbm,
          pack_v, d00, d01, d10, d11, ob0, ob1,
          sg00, sg01, sg10, sg11, so0, so1):
    wid = lax.axis_index("s") * NC + lax.axis_index("c")
    ns = pl.ds(wid * CHUNK, CHUNK)

    pltpu.sync_copy(pack_hbm, pack_v)

    idxv = plsc.bitcast(pack_v[2, pl.ds(0, LANES)], jnp.int32)
    selv = plsc.bitcast(pack_v[3, pl.ds(0, LANES)], jnp.int32)
    lane = lax.broadcasted_iota(jnp.int32, (LANES,), 0)

    def extract(vec, m):
        s = jnp.int32(0)
        for k in range(LBITS):
            bit = jnp.any(m & (((vec >> k) & 1) == 1))
            s = s + jnp.where(bit, jnp.int32(1 << k), jnp.int32(0))
        return s

    i0s, i1s, s0s, s1s = [], [], [], []
    for b in range(B):
        m0 = lane == b
        m1 = lane == (B + b)
        i0s.append(extract(idxv, m0))
        i1s.append(extract(idxv, m1))
        s0s.append(jnp.any(m0 & (selv == 1)))
        s1s.append(jnp.any(m1 & (selv == 1)))

    d0bufs = (d00, d01)
    d1bufs = (d10, d11)
    obufs = (ob0, ob1)
    g0sems = (sg00, sg01)
    g1sems = (sg10, sg11)
    osems = (so0, so1)

    def issue(b):
        p = b % 2
        d0, d1 = d0bufs[p], d1bufs[p]

        @pl.when(s0s[b])
        def _():
            pltpu.async_copy(valueT_hbm.at[:, ns], d0, g0sems[p])

        @pl.when(jnp.logical_not(s0s[b]))
        def _():
            pltpu.async_copy(histT_hbm.at[i0s[b], :, ns], d0, g0sems[p])

        @pl.when(s1s[b])
        def _():
            pltpu.async_copy(valueT_hbm.at[:, ns], d1, g1sems[p])

        @pl.when(jnp.logical_not(s1s[b]))
        def _():
            pltpu.async_copy(histT_hbm.at[i1s[b], :, ns], d1, g1sems[p])

    issue(0)
    out_pending = [None, None]

    for b in range(B):
        p = b % 2
        if b < B - 1:
            issue(b + 1)
        pltpu.make_async_copy(histT_hbm.at[0, :, ns], d0bufs[p], g0sems[p]).wait()
        pltpu.make_async_copy(histT_hbm.at[0, :, ns], d1bufs[p], g1sems[p]).wait()
        if out_pending[p] is not None:
            out_pending[p].wait()
            out_pending[p] = None

        d0, d1, ob = d0bufs[p], d1bufs[p], obufs[p]
        bs = pl.ds(b * LANES, LANES)
        wa = pack_v[0, bs]
        wb = pack_v[1, bs]

        def compute(r, carry):
            for g in range(GPR):
                sl = pl.ds(g * LANES, LANES)
                a = d0[r, sl]
                c = d1[r, sl]
                ob[r, sl] = wa * a + wb * c
            return carry

        lax.fori_loop(0, D, compute, 0)

        out_pending[p] = pltpu.async_copy(ob, outT_hbm.at[b, :, ns], osems[p])

    for p in range(2):
        if out_pending[p] is not None:
            out_pending[p].wait()


@jax.jit
def _sc_delay(histT, valueT, pack):
    call = functools.partial(
        pl.kernel,
        mesh=plsc.VectorSubcoreMesh(core_axis_name="c", subcore_axis_name="s"),
        compiler_params=pltpu.CompilerParams(needs_layout_passes=False),
        out_type=jax.ShapeDtypeStruct((B, D, N), jnp.float32),
        scratch_types=[
            pltpu.VMEM((8, 128), jnp.float32),
            pltpu.VMEM((D, CHUNK), jnp.float32),
            pltpu.VMEM((D, CHUNK), jnp.float32),
            pltpu.VMEM((D, CHUNK), jnp.float32),
            pltpu.VMEM((D, CHUNK), jnp.float32),
            pltpu.VMEM((D, CHUNK), jnp.float32),
            pltpu.VMEM((D, CHUNK), jnp.float32),
            pltpu.SemaphoreType.DMA,
            pltpu.SemaphoreType.DMA,
            pltpu.SemaphoreType.DMA,
            pltpu.SemaphoreType.DMA,
            pltpu.SemaphoreType.DMA,
            pltpu.SemaphoreType.DMA,
        ],
    )(_body)
    return call(histT, valueT, pack)


def kernel(history, value, delay_frac, write_idx, delay_int):
    histT = history.transpose(0, 2, 1)
    valueT = value.transpose(1, 0)

    wi = jnp.asarray(write_idx, jnp.int32)
    i0 = jnp.mod(wi - delay_int, L)
    i1 = jnp.mod(wi - delay_int - 1, L)
    wrow = jnp.mod(wi, L)
    w = delay_frac.astype(jnp.float32)
    ids = jnp.concatenate([i0, i1]).astype(jnp.int32)
    sel = jnp.concatenate([i0 == wrow, i1 == wrow]).astype(jnp.int32)

    pack = jnp.zeros((8, 128), jnp.float32)
    pack = pack.at[0].set(jnp.repeat(1.0 - w, LANES))
    pack = pack.at[1].set(jnp.repeat(w, LANES))
    pack = pack.at[2, :LANES].set(jax.lax.bitcast_convert_type(ids, jnp.float32))
    pack = pack.at[3, :LANES].set(jax.lax.bitcast_convert_type(sel, jnp.float32))

    outT = _sc_delay(histT, valueT, pack)
    return outT.transpose(0, 2, 1)


# speedup vs baseline: 17.5539x; 1.0308x over previous
import functools

import jax
import jax.numpy as jnp
from jax import lax
from jax.experimental import pallas as pl
from jax.experimental.pallas import tpu as pltpu
from jax.experimental.pallas import tpu_sc as plsc

L = 64
N = 16384
D = 16
B = 8

NC = 2
NS = 16
NW = NC * NS
HALF = N // 2
LANES = 16
RW = 128
GRP = RW // LANES
NR = HALF // RW
LBITS = 6


def _body(histT_hbm, valueT_hbm, pack_hbm, outT_hbm,
          pack_v, d00, d01, d10, d11, ob0, ob1,
          sg00, sg01, sg10, sg11, so0, so1):
    wid = lax.axis_index("s") * NC + lax.axis_index("c")
    dw = wid // 2
    nh = pl.ds((wid % 2) * HALF, HALF)

    pltpu.sync_copy(pack_hbm, pack_v)

    idxv = plsc.bitcast(pack_v[2, pl.ds(0, LANES)], jnp.int32)
    selv = plsc.bitcast(pack_v[3, pl.ds(0, LANES)], jnp.int32)
    lane = lax.broadcasted_iota(jnp.int32, (LANES,), 0)

    def extract(vec, m):
        s = jnp.int32(0)
        for k in range(LBITS):
            bit = jnp.any(m & (((vec >> k) & 1) == 1))
            s = s + jnp.where(bit, jnp.int32(1 << k), jnp.int32(0))
        return s

    i0s, i1s, s0s, s1s = [], [], [], []
    for b in range(B):
        m0 = lane == b
        m1 = lane == (B + b)
        i0s.append(extract(idxv, m0))
        i1s.append(extract(idxv, m1))
        s0s.append(jnp.any(m0 & (selv == 1)))
        s1s.append(jnp.any(m1 & (selv == 1)))

    d0bufs = (d00, d01)
    d1bufs = (d10, d11)
    obufs = (ob0, ob1)
    g0sems = (sg00, sg01)
    g1sems = (sg10, sg11)
    osems = (so0, so1)

    def issue(b):
        p = b % 2
        d0, d1 = d0bufs[p], d1bufs[p]

        @pl.when(s0s[b])
        def _():
            pltpu.async_copy(valueT_hbm.at[dw, nh], d0, g0sems[p])

        @pl.when(jnp.logical_not(s0s[b]))
        def _():
            pltpu.async_copy(histT_hbm.at[i0s[b], dw, nh], d0, g0sems[p])

        @pl.when(s1s[b])
        def _():
            pltpu.async_copy(valueT_hbm.at[dw, nh], d1, g1sems[p])

        @pl.when(jnp.logical_not(s1s[b]))
        def _():
            pltpu.async_copy(histT_hbm.at[i1s[b], dw, nh], d1, g1sems[p])

    issue(0)
    out_pending = [None, None]

    for b in range(B):
        p = b % 2
        if b < B - 1:
            issue(b + 1)
        pltpu.make_async_copy(histT_hbm.at[0, dw, nh], d0bufs[p], g0sems[p]).wait()
        pltpu.make_async_copy(histT_hbm.at[0, dw, nh], d1bufs[p], g1sems[p]).wait()
        if out_pending[p] is not None:
            out_pending[p].wait()
            out_pending[p] = None

        d0, d1, ob = d0bufs[p], d1bufs[p], obufs[p]
        bs = pl.ds(b * LANES, LANES)
        wa = pack_v[0, bs]
        wb = pack_v[1, bs]

        def compute(r, carry):
            base = r * RW
            for g in range(GRP):
                sl = pl.ds(base + g * LANES, LANES)
                ob[sl] = wa * d0[sl] + wb * d1[sl]
            return carry

        lax.fori_loop(0, NR, compute, 0)

        out_pending[p] = pltpu.async_copy(ob, outT_hbm.at[b, dw, nh], osems[p])

    for p in range(2):
        if out_pending[p] is not None:
            out_pending[p].wait()


@jax.jit
def _sc_delay(histT, valueT, pack):
    call = functools.partial(
        pl.kernel,
        mesh=plsc.VectorSubcoreMesh(core_axis_name="c", subcore_axis_name="s"),
        compiler_params=pltpu.CompilerParams(needs_layout_passes=False),
        out_type=jax.ShapeDtypeStruct((B, D, N), jnp.float32),
        scratch_types=[
            pltpu.VMEM((4, 128), jnp.float32),
            pltpu.VMEM((HALF,), jnp.float32),
            pltpu.VMEM((HALF,), jnp.float32),
            pltpu.VMEM((HALF,), jnp.float32),
            pltpu.VMEM((HALF,), jnp.float32),
            pltpu.VMEM((HALF,), jnp.float32),
            pltpu.VMEM((HALF,), jnp.float32),
            pltpu.SemaphoreType.DMA,
            pltpu.SemaphoreType.DMA,
            pltpu.SemaphoreType.DMA,
            pltpu.SemaphoreType.DMA,
            pltpu.SemaphoreType.DMA,
            pltpu.SemaphoreType.DMA,
        ],
    )(_body)
    return call(histT, valueT, pack)


def kernel(history, value, delay_frac, write_idx, delay_int):
    histT = history.transpose(0, 2, 1)
    valueT = value.transpose(1, 0)

    wi = jnp.asarray(write_idx, jnp.int32)
    i0 = jnp.mod(wi - delay_int, L)
    i1 = jnp.mod(wi - delay_int - 1, L)
    wrow = jnp.mod(wi, L)
    w = delay_frac.astype(jnp.float32)
    ids = jnp.concatenate([i0, i1]).astype(jnp.int32)
    sel = jnp.concatenate([i0 == wrow, i1 == wrow]).astype(jnp.int32)

    pad = jnp.zeros((112,), jnp.float32)
    pack = jnp.stack([
        jnp.repeat(1.0 - w, LANES),
        jnp.repeat(w, LANES),
        jnp.concatenate([jax.lax.bitcast_convert_type(ids, jnp.float32), pad]),
        jnp.concatenate([jax.lax.bitcast_convert_type(sel, jnp.float32), pad]),
    ])

    outT = _sc_delay(histT, valueT, pack)
    return outT.transpose(0, 2, 1)
